# baseline (speedup 1.0000x reference)
import functools

import jax
import jax.numpy as jnp
from jax import lax
from jax.experimental import pallas as pl
from jax.experimental.pallas import tpu as pltpu
from jax.experimental.pallas import tpu_sc as plsc

EMBED_DIM = 64
CHUNK = 128
K = 5


@jax.jit
def _embed(idx3, weight):
    info = plsc.get_sparse_core_info()
    nw = info.num_cores * info.num_subcores
    n_chunks = idx3.shape[1]
    per_w = n_chunks * CHUNK
    n = nw * per_w
    n_megas = n_chunks // K
    mega_rows = K * CHUNK

    mesh = plsc.VectorSubcoreMesh(core_axis_name="c", subcore_axis_name="s")
    NB = 3

    @functools.partial(
        pl.kernel,
        mesh=mesh,
        compiler_params=pltpu.CompilerParams(use_tc_tiling_on_sc=False),
        out_type=jax.ShapeDtypeStruct((n, EMBED_DIM), jnp.float32),
        scratch_types=[
            pltpu.VMEM((n_chunks, CHUNK), jnp.int32),
            pltpu.VMEM((NB, mega_rows, EMBED_DIM), jnp.float32),
            pltpu.SemaphoreType.DMA,
            pltpu.SemaphoreType.DMA,
            pltpu.SemaphoreType.DMA,
            pltpu.SemaphoreType.DMA,
            pltpu.SemaphoreType.DMA,
            pltpu.SemaphoreType.DMA,
        ],
    )
    def emb(idx_hbm, table_hbm, out_hbm, idx_v, rows_v, g0, g1, g2, s0, s1, s2):
        wid = lax.axis_index("c") * info.num_subcores + lax.axis_index("s")
        base = wid * per_w
        pltpu.sync_copy(idx_hbm.at[wid], idx_v)

        gsem = (g0, g1, g2)
        ssem = (s0, s1, s2)

        def gath(m, k, b):
            return pltpu.make_async_copy(
                table_hbm.at[idx_v.at[m * K + k]],
                rows_v.at[b].at[pl.ds(k * CHUNK, CHUNK)],
                gsem[b],
            )

        def fire(m, b):
            for k in range(K):
                gath(m, k, b).start()

        def drain(m, b):
            for k in range(K):
                gath(m, k, b).wait()

        def scat(m, b):
            return pltpu.make_async_copy(
                rows_v.at[b],
                out_hbm.at[pl.ds(base + m * mega_rows, mega_rows)],
                ssem[b],
            )

        for b in range(NB):
            fire(b, b)

        for m in range(n_megas):
            b = m % NB
            drain(m, b)
            scat(m, b).start()
            if m + NB < n_megas:
                scat(m, b).wait()
                fire(m + NB, b)

        for m in range(n_megas - NB, n_megas):
            scat(m, m % NB).wait()

    return emb(idx3, weight)


def kernel(input, weight):
    b, h = input.shape
    n = b * h
    info = plsc.get_sparse_core_info()
    nw = info.num_cores * info.num_subcores
    idx3 = input.reshape(nw, n // (nw * CHUNK), CHUNK).astype(jnp.int32)
    out = _embed(idx3, weight)
    return out.reshape(b, h, weight.shape[1])
